# baseline (speedup 1.0000x reference)
import jax
import jax.numpy as jnp
from jax.experimental import pallas as pl
from jax.experimental.pallas import tpu as pltpu

_SAMPLE = 8
_BLK = 16384


def _reduce_kernel(x_ref, sums_ref, acc_ref, total_rows: int):
    step = pl.program_id(0)
    nsteps = pl.num_programs(0)
    blk = x_ref.shape[0]

    @pl.when(step == 0)
    def _init():
        acc_ref[0] = 0.0
        acc_ref[1] = 0.0

    @pl.when(step < nsteps - 1)
    def _full():
        x = x_ref[...]
        acc_ref[0] += jnp.sum(x)
        acc_ref[1] += jnp.sum(x * x)

    @pl.when(step == nsteps - 1)
    def _last():
        valid = total_rows - step * blk
        row = jax.lax.broadcasted_iota(jnp.int32, x_ref.shape, 0)
        x = jnp.where(row < valid, x_ref[...], 0.0)
        acc_ref[0] += jnp.sum(x)
        acc_ref[1] += jnp.sum(x * x)
        sums_ref[0] = acc_ref[0]
        sums_ref[1] = acc_ref[1]


def _gather_kernel(idx_ref, x_ref, y_ref, sums_ref, xs_ref, ysum_ref,
                   n_total: float):
    step = pl.program_id(0)

    @pl.when(step == 0)
    def _init():
        ysum_ref[...] = jnp.zeros_like(ysum_ref)

    mean = sums_ref[0] / n_total
    var = sums_ref[1] / n_total - mean * mean
    inv_std = jax.lax.rsqrt(var)
    xs_ref[...] = (x_ref[...] - mean) * inv_std
    ysum_ref[...] += y_ref[...].reshape(1, 1)


def kernel(x_train, y_train, indices):
    n, h, w = x_train.shape
    f = h * w
    total = n * f
    assert total % 128 == 0
    rows128 = total // 128
    xflat = x_train.reshape(rows128, 128)
    grid1 = (rows128 + _BLK - 1) // _BLK

    import functools

    sums = pl.pallas_call(
        functools.partial(_reduce_kernel, total_rows=rows128),
        grid=(grid1,),
        in_specs=[pl.BlockSpec((_BLK, 128), lambda i: (i, 0))],
        out_specs=pl.BlockSpec(memory_space=pltpu.SMEM),
        out_shape=jax.ShapeDtypeStruct((2,), jnp.float32),
        scratch_shapes=[pltpu.SMEM((2,), jnp.float32)],
    )(xflat)

    y3 = y_train.reshape(n, 1, 1)
    grid_spec = pltpu.PrefetchScalarGridSpec(
        num_scalar_prefetch=1,
        grid=(_SAMPLE,),
        in_specs=[
            pl.BlockSpec((1, h, w), lambda i, idx: (idx[i], 0, 0)),
            pl.BlockSpec((1, 1, 1), lambda i, idx: (idx[i], 0, 0)),
            pl.BlockSpec(memory_space=pltpu.SMEM),
        ],
        out_specs=[
            pl.BlockSpec((1, h, w), lambda i, idx: (i, 0, 0)),
            pl.BlockSpec((1, 1), lambda i, idx: (0, 0)),
        ],
    )
    xs, ysum = pl.pallas_call(
        functools.partial(_gather_kernel, n_total=float(total)),
        grid_spec=grid_spec,
        out_shape=[
            jax.ShapeDtypeStruct((_SAMPLE, h, w), jnp.float32),
            jax.ShapeDtypeStruct((1, 1), y_train.dtype),
        ],
    )(indices, x_train, y3, sums)
    return xs, ysum[0, 0]


# speedup vs baseline: 1.5470x; 1.5470x over previous
import jax
import jax.numpy as jnp
from jax.experimental import pallas as pl
from jax.experimental.pallas import tpu as pltpu

_SAMPLE = 8
_ROWS = 1500


def _pass_kernel(idx_ref, x_ref, y_ref, xs_ref, ysum_ref, s_ref, sq_ref):
    step = pl.program_id(0)
    nsteps = pl.num_programs(0)
    rows = x_ref.shape[0]

    @pl.when(step == 0)
    def _init():
        s_ref[...] = jnp.zeros_like(s_ref)
        sq_ref[...] = jnp.zeros_like(sq_ref)
        ysum_ref[...] = jnp.zeros_like(ysum_ref)

    x = x_ref[...]
    s_ref[...] += jnp.sum(x, axis=0)
    sq_ref[...] += jnp.sum(x * x, axis=0)

    base = step * rows
    for j in range(_SAMPLE):
        idx = idx_ref[j]
        local = idx - base

        @pl.when((idx >= base) & (idx < base + rows))
        def _copy():
            xs_ref[pl.ds(j, 1), :, :] = x_ref[pl.ds(local, 1), :, :]
            ysum_ref[...] += y_ref[0, pl.ds(local, 1), :]

    @pl.when(step == nsteps - 1)
    def _final():
        h, w = s_ref.shape
        total = jnp.float32(h * w * rows) * jnp.float32(nsteps)
        mean = jnp.sum(s_ref[...]) / total
        var = jnp.sum(sq_ref[...]) / total - mean * mean
        inv_std = jax.lax.rsqrt(var)
        xs_ref[...] = (xs_ref[...] - mean) * inv_std


def kernel(x_train, y_train, indices):
    n, h, w = x_train.shape
    assert n % _ROWS == 0
    grid = n // _ROWS
    y2 = y_train.reshape(grid, _ROWS, 1)

    xs, ysum = pl.pallas_call(
        _pass_kernel,
        grid=(grid,),
        in_specs=[
            pl.BlockSpec(memory_space=pltpu.SMEM),
            pl.BlockSpec((_ROWS, h, w), lambda i: (i, 0, 0)),
            pl.BlockSpec((1, _ROWS, 1), lambda i: (i, 0, 0)),
        ],
        out_specs=[
            pl.BlockSpec((_SAMPLE, h, w), lambda i: (0, 0, 0)),
            pl.BlockSpec((1, 1), lambda i: (0, 0)),
        ],
        out_shape=[
            jax.ShapeDtypeStruct((_SAMPLE, h, w), jnp.float32),
            jax.ShapeDtypeStruct((1, 1), y_train.dtype),
        ],
        scratch_shapes=[
            pltpu.VMEM((h, w), jnp.float32),
            pltpu.VMEM((h, w), jnp.float32),
        ],
    )(indices, x_train, y2)
    return xs, ysum[0, 0]


# speedup vs baseline: 21.2253x; 13.7203x over previous
import jax
import jax.numpy as jnp
from jax.experimental import pallas as pl
from jax.experimental.pallas import tpu as pltpu

_SAMPLE = 8
_CHUNK = 3072


def _pass_kernel(idx_ref, x_ref, y_ref, xs_ref, ysum_ref, s_ref, sq_ref,
                 n_images: int):
    step = pl.program_id(0)
    nsteps = pl.num_programs(0)
    chunk = x_ref.shape[2]
    base = step * chunk

    @pl.when(step == 0)
    def _init():
        s_ref[...] = jnp.zeros_like(s_ref)
        sq_ref[...] = jnp.zeros_like(sq_ref)

    @pl.when(step < nsteps - 1)
    def _full():
        x = x_ref[...]
        s_ref[...] += jnp.sum(x, axis=0)
        sq_ref[...] += jnp.sum(x * x, axis=0)

    lane = jax.lax.broadcasted_iota(jnp.int32, x_ref.shape, 2)
    out_lane = jax.lax.broadcasted_iota(jnp.int32, xs_ref.shape, 2)
    for j in range(_SAMPLE):
        idx = idx_ref[j]
        local = idx - base

        @pl.when((idx >= base) & (idx < base + chunk))
        def _copy():
            img = jnp.sum(jnp.where(lane == local, x_ref[...], 0.0), axis=2,
                          keepdims=True)
            bcast = jnp.broadcast_to(img, xs_ref.shape)
            xs_ref[...] = jnp.where(out_lane == j, bcast, xs_ref[...])

    @pl.when(step == nsteps - 1)
    def _last():
        valid = n_images - base
        x = jnp.where(lane < valid, x_ref[...], 0.0)
        s = s_ref[...] + jnp.sum(x, axis=0)
        sq = sq_ref[...] + jnp.sum(x * x, axis=0)

        total = jnp.float32(x_ref.shape[0] * x_ref.shape[1]) * n_images
        mean = jnp.sum(s) / total
        var = jnp.sum(sq) / total - mean * mean
        inv_std = jax.lax.rsqrt(var)
        xs_ref[...] = (xs_ref[...] - mean) * inv_std

        yv = y_ref[...]
        ylane = jax.lax.broadcasted_iota(jnp.int32, yv.shape, 1)
        hits = jnp.zeros_like(yv)
        for j in range(_SAMPLE):
            hits += jnp.where(ylane == idx_ref[j], 1, 0)
        ysum_ref[0, 0] = jnp.sum(yv * hits)


def kernel(x_train, y_train, indices):
    n, h, w = x_train.shape
    xt = x_train.transpose(1, 2, 0)
    y2 = y_train.reshape(1, n)
    grid = (n + _CHUNK - 1) // _CHUNK

    import functools

    xs, ysum = pl.pallas_call(
        functools.partial(_pass_kernel, n_images=n),
        grid=(grid,),
        in_specs=[
            pl.BlockSpec(memory_space=pltpu.SMEM),
            pl.BlockSpec((h, w, _CHUNK), lambda i: (0, 0, i)),
            pl.BlockSpec((1, n), lambda i: (0, 0)),
        ],
        out_specs=[
            pl.BlockSpec((h, w, _SAMPLE), lambda i: (0, 0, 0)),
            pl.BlockSpec(memory_space=pltpu.SMEM),
        ],
        out_shape=[
            jax.ShapeDtypeStruct((h, w, _SAMPLE), jnp.float32),
            jax.ShapeDtypeStruct((1, 1), y_train.dtype),
        ],
        scratch_shapes=[
            pltpu.VMEM((w, _CHUNK), jnp.float32),
            pltpu.VMEM((w, _CHUNK), jnp.float32),
        ],
    )(indices, xt, y2)
    return xs.transpose(2, 0, 1), ysum[0, 0]
